# baseline (speedup 1.0000x reference)
import functools

import jax
import jax.numpy as jnp
from jax import lax
from jax.experimental import pallas as pl
from jax.experimental.pallas import tpu as pltpu
from jax.experimental.pallas import tpu_sc as plsc

_N = 4096
_D = 32
_NEG = 5
_NC = 2
_NS = 16
_NW = _NC * _NS
_L = 16
_V = 1000000
_POS_W = _N // _NW
_NEG_W = _N * _NEG // _NW
_RING = 16


def _sc_scores(u_t, v_t, pos_u, pos_v, neg_v):
    mesh = plsc.VectorSubcoreMesh(core_axis_name="c", subcore_axis_name="s")

    @functools.partial(
        pl.kernel,
        mesh=mesh,
        compiler_params=pltpu.CompilerParams(needs_layout_passes=False),
        out_type=[
            jax.ShapeDtypeStruct((_N,), jnp.float32),
            jax.ShapeDtypeStruct((_N * _NEG,), jnp.float32),
            jax.ShapeDtypeStruct((_D, 128), jnp.float32),
        ],
        scratch_types=[
            pltpu.VMEM((_NEG_W,), jnp.int32),
            pltpu.VMEM((_RING, _D, 128), jnp.float32),
            pltpu.VMEM((_D, _POS_W), jnp.float32),
            pltpu.VMEM((_POS_W,), jnp.float32),
            pltpu.VMEM((_NEG_W,), jnp.float32),
        ] + [pltpu.SemaphoreType.DMA] * _RING,
    )
    def k(u_hbm, v_hbm, pu_hbm, pv_hbm, nv_hbm, out_s, out_ns, dummy,
          sm_idx, ring, cu_t, s_v, ns_v, *sems):
        wid = lax.axis_index("s") * _NC + lax.axis_index("c")
        base = wid * _POS_W
        lanes = lax.iota(jnp.int32, _L)
        lanes_hi = lanes + _L

        def sread(i):
            return plsc.load_gather(sm_idx, [jnp.full((_L,), i, jnp.int32)])[0]

        def col_copy(tab, r_scalar, t, sem):
            off = pl.multiple_of((r_scalar >> 7) * 128, 128)
            pltpu.async_copy(tab.at[:, pl.ds(off, 128)], ring.at[t], sem)

        def drain(t, sem):
            pltpu.make_async_copy(dummy, ring.at[t], sem).wait()

        def extract(t, cvec):
            tvec = jnp.full((_L,), t, jnp.int32)
            lo = plsc.load_gather(ring, [tvec, lanes, cvec])
            hi = plsc.load_gather(ring, [tvec, lanes_hi, cvec])
            return lo, hi

        def run_phase(tab, nrows, process):
            for t in range(_RING):
                col_copy(tab, sread(t), t, sems[t])

            def body(q, carry):
                for t in range(_RING):
                    row = q * _RING + t
                    drain(t, sems[t])
                    rs = sread(row)
                    cvec = jnp.full((_L,), rs & 127, jnp.int32)
                    process(row, extract(t, cvec))
                    nxt = row + _RING

                    @pl.when(nxt < nrows)
                    def _():
                        col_copy(tab, sread(nxt), t, sems[t])

                return carry

            lax.fori_loop(0, nrows // _RING, body, 0, unroll=False)

        pltpu.sync_copy(pu_hbm.at[pl.ds(base, _POS_W)],
                        sm_idx.at[pl.ds(0, _POS_W)])

        def proc_u(row, vals):
            lo, hi = vals
            rvec = jnp.full((_L,), row, jnp.int32)
            plsc.store_scatter(cu_t, [lanes, rvec], lo)
            plsc.store_scatter(cu_t, [lanes_hi, rvec], hi)

        run_phase(u_hbm, _POS_W, proc_u)

        pltpu.sync_copy(pv_hbm.at[pl.ds(base, _POS_W)],
                        sm_idx.at[pl.ds(0, _POS_W)])

        def proc_v(row, vals):
            lo, hi = vals
            rvec = jnp.full((_L,), row, jnp.int32)
            ul = plsc.load_gather(cu_t, [lanes, rvec])
            uh = plsc.load_gather(cu_t, [lanes_hi, rvec])
            s = jnp.sum(lo * ul + hi * uh)
            plsc.store_scatter(s_v, [rvec], jnp.full((_L,), s, jnp.float32),
                               mask=lanes == 0)

        run_phase(v_hbm, _POS_W, proc_v)

        pltpu.sync_copy(nv_hbm.at[pl.ds(wid * _NEG_W, _NEG_W)],
                        sm_idx.at[pl.ds(0, _NEG_W)])

        def proc_n(row, vals):
            lo, hi = vals
            uvec = jnp.full((_L,), row // _NEG, jnp.int32)
            ul = plsc.load_gather(cu_t, [lanes, uvec])
            uh = plsc.load_gather(cu_t, [lanes_hi, uvec])
            s = jnp.sum(lo * ul + hi * uh)
            plsc.store_scatter(ns_v, [jnp.full((_L,), row, jnp.int32)],
                               jnp.full((_L,), s, jnp.float32),
                               mask=lanes == 0)

        run_phase(v_hbm, _NEG_W, proc_n)

        pltpu.sync_copy(s_v, out_s.at[pl.ds(base, _POS_W)])
        pltpu.sync_copy(ns_v, out_ns.at[pl.ds(wid * _NEG_W, _NEG_W)])

    score, nscore, _ = k(u_t, v_t, pos_u, pos_v, neg_v)
    return score, nscore


def _log_sigmoid(x):
    return jnp.minimum(x, 0.0) - jnp.log(1.0 + jnp.exp(-jnp.abs(x)))


def _tc_loss_body(s_ref, ns_ref, out_ref):
    total = jnp.sum(_log_sigmoid(s_ref[...]))
    total += jnp.sum(_log_sigmoid(-ns_ref[...]))
    out_ref[0, 0] = total


def _tc_loss(score2d, nscore2d):
    return pl.pallas_call(
        _tc_loss_body,
        out_shape=jax.ShapeDtypeStruct((1, 1), jnp.float32),
        out_specs=pl.BlockSpec(memory_space=pltpu.SMEM),
    )(score2d, nscore2d)


def kernel(pos_u, pos_v, neg_v, batch_size, u_emb, v_emb):
    pu = pos_u.reshape(_N)
    pv = pos_v.reshape(_N)
    nv = neg_v.reshape(_N * _NEG)
    score, nscore = _sc_scores(u_emb.T, v_emb.T, pu, pv, nv)
    total = _tc_loss(score.reshape(_N // 128, 128),
                     nscore.reshape(_N * _NEG // 128, 128))[0, 0]
    return -total / batch_size


# speedup vs baseline: 1.0414x; 1.0414x over previous
import functools

import jax
import jax.numpy as jnp
from jax import lax
from jax.experimental import pallas as pl
from jax.experimental.pallas import tpu as pltpu
from jax.experimental.pallas import tpu_sc as plsc

_N = 4096
_D = 32
_NEG = 5
_NC = 2
_NS = 16
_NW = _NC * _NS
_L = 16
_V = 1000000
_POS_W = _N // _NW
_NEG_W = _N * _NEG // _NW
_RING = 8


def _sc_scores(u_t, v_t, pos_u, pos_v, neg_v):
    mesh = plsc.VectorSubcoreMesh(core_axis_name="c", subcore_axis_name="s")

    @functools.partial(
        pl.kernel,
        mesh=mesh,
        compiler_params=pltpu.CompilerParams(needs_layout_passes=False),
        out_type=[
            jax.ShapeDtypeStruct((_N,), jnp.float32),
            jax.ShapeDtypeStruct((_N * _NEG,), jnp.float32),
            jax.ShapeDtypeStruct((4, 8, 128), jnp.float32),
        ],
        scratch_types=[
            pltpu.VMEM((_NEG_W,), jnp.int32),
            pltpu.VMEM((_RING, 4, 8, 128), jnp.float32),
            pltpu.VMEM((_D, _POS_W), jnp.float32),
            pltpu.VMEM((_POS_W,), jnp.float32),
            pltpu.VMEM((_NEG_W,), jnp.float32),
        ] + [pltpu.SemaphoreType.DMA] * _RING,
    )
    def k(u_hbm, v_hbm, pu_hbm, pv_hbm, nv_hbm, out_s, out_ns, dummy,
          sm_idx, ring, cu_t, s_v, ns_v, *sems):
        wid = lax.axis_index("s") * _NC + lax.axis_index("c")
        base = wid * _POS_W
        lanes = lax.iota(jnp.int32, _L)
        lanes_hi = lanes + _L

        def sread(i):
            return plsc.load_gather(sm_idx, [jnp.full((_L,), i, jnp.int32)])[0]

        def col_copy(tab, r_scalar, t, sem):
            off = pl.multiple_of((r_scalar >> 7) * 128, 128)
            for i in range(4):
                pltpu.async_copy(tab.at[i, :, pl.ds(off, 128)],
                                 ring.at[t, i], sem)

        def drain(t, sem):
            pltpu.make_async_copy(dummy, ring.at[t], sem).wait()

        iv_lo = lanes >> 3
        iv_hi = lanes_hi >> 3
        sv = lanes & 7

        def extract(t, cvec):
            tvec = jnp.full((_L,), t, jnp.int32)
            lo = plsc.load_gather(ring, [tvec, iv_lo, sv, cvec])
            hi = plsc.load_gather(ring, [tvec, iv_hi, sv, cvec])
            return lo, hi

        def run_phase(tab, nrows, process):
            for t in range(_RING):
                col_copy(tab, sread(t), t, sems[t])

            def body(q, carry):
                for t in range(_RING):
                    row = q * _RING + t
                    drain(t, sems[t])
                    rs = sread(row)
                    cvec = jnp.full((_L,), rs & 127, jnp.int32)
                    process(row, extract(t, cvec))
                    nxt = row + _RING

                    @pl.when(nxt < nrows)
                    def _():
                        col_copy(tab, sread(nxt), t, sems[t])

                return carry

            lax.fori_loop(0, nrows // _RING, body, 0, unroll=False)

        pltpu.sync_copy(pu_hbm.at[pl.ds(base, _POS_W)],
                        sm_idx.at[pl.ds(0, _POS_W)])

        def proc_u(row, vals):
            lo, hi = vals
            rvec = jnp.full((_L,), row, jnp.int32)
            plsc.store_scatter(cu_t, [lanes, rvec], lo)
            plsc.store_scatter(cu_t, [lanes_hi, rvec], hi)

        run_phase(u_hbm, _POS_W, proc_u)

        pltpu.sync_copy(pv_hbm.at[pl.ds(base, _POS_W)],
                        sm_idx.at[pl.ds(0, _POS_W)])

        def proc_v(row, vals):
            lo, hi = vals
            rvec = jnp.full((_L,), row, jnp.int32)
            ul = plsc.load_gather(cu_t, [lanes, rvec])
            uh = plsc.load_gather(cu_t, [lanes_hi, rvec])
            s = jnp.sum(lo * ul + hi * uh)
            plsc.store_scatter(s_v, [rvec], jnp.full((_L,), s, jnp.float32),
                               mask=lanes == 0)

        run_phase(v_hbm, _POS_W, proc_v)

        pltpu.sync_copy(nv_hbm.at[pl.ds(wid * _NEG_W, _NEG_W)],
                        sm_idx.at[pl.ds(0, _NEG_W)])

        def proc_n(row, vals):
            lo, hi = vals
            uvec = jnp.full((_L,), row // _NEG, jnp.int32)
            ul = plsc.load_gather(cu_t, [lanes, uvec])
            uh = plsc.load_gather(cu_t, [lanes_hi, uvec])
            s = jnp.sum(lo * ul + hi * uh)
            plsc.store_scatter(ns_v, [jnp.full((_L,), row, jnp.int32)],
                               jnp.full((_L,), s, jnp.float32),
                               mask=lanes == 0)

        run_phase(v_hbm, _NEG_W, proc_n)

        pltpu.sync_copy(s_v, out_s.at[pl.ds(base, _POS_W)])
        pltpu.sync_copy(ns_v, out_ns.at[pl.ds(wid * _NEG_W, _NEG_W)])

    score, nscore, _ = k(u_t.reshape(4, 8, _V), v_t.reshape(4, 8, _V),
                         pos_u, pos_v, neg_v)
    return score, nscore


def _log_sigmoid(x):
    return jnp.minimum(x, 0.0) - jnp.log(1.0 + jnp.exp(-jnp.abs(x)))


def _tc_loss_body(s_ref, ns_ref, out_ref):
    total = jnp.sum(_log_sigmoid(s_ref[...]))
    total += jnp.sum(_log_sigmoid(-ns_ref[...]))
    out_ref[0, 0] = total


def _tc_loss(score2d, nscore2d):
    return pl.pallas_call(
        _tc_loss_body,
        out_shape=jax.ShapeDtypeStruct((1, 1), jnp.float32),
        out_specs=pl.BlockSpec(memory_space=pltpu.SMEM),
    )(score2d, nscore2d)


def kernel(pos_u, pos_v, neg_v, batch_size, u_emb, v_emb):
    pu = pos_u.reshape(_N)
    pv = pos_v.reshape(_N)
    nv = neg_v.reshape(_N * _NEG)
    score, nscore = _sc_scores(u_emb.T, v_emb.T, pu, pv, nv)
    total = _tc_loss(score.reshape(_N // 128, 128),
                     nscore.reshape(_N * _NEG // 128, 128))[0, 0]
    return -total / batch_size
